# baseline (speedup 1.0000x reference)
import jax
import jax.numpy as jnp
from jax.experimental import pallas as pl

B, N, E, D, H = 8, 64, 256, 512, 8
HID = D // H
NEG = 0.1
G = 1
PB = B // G


def _leaky(x):
    return jnp.where(x >= 0, x, NEG * x)


def _fused_kernel(obj_ref, attr_ref, rela_ref, edges_ref,
                  W_attr_ref, b_attr_ref, W_rela_ref, b_rela_ref,
                  W_lin_ref, W_edge_ref, W_attn_ref,
                  out_ref, attr3_ref, rela3_ref):
    f32 = jnp.float32
    obj2 = obj_ref[...].reshape(PB * N, D)
    attr2 = attr_ref[...].reshape(PB * N, D)
    rela2 = rela_ref[...].reshape(PB * E, D)

    new_attr = jax.nn.relu(
        jnp.dot(obj2, W_attr_ref[0:D, :], preferred_element_type=f32)
        + jnp.dot(attr2, W_attr_ref[D:2 * D, :], preferred_element_type=f32)
        + b_attr_ref[...]) + attr2
    attr3_ref[...] = new_attr.reshape(PB, N, D)

    objWr1 = jnp.dot(obj2, W_rela_ref[0:D, :],
                     preferred_element_type=f32)
    iota_en = jax.lax.broadcasted_iota(jnp.int32, (E, N), 1)
    relaWr2 = jnp.dot(rela2, W_rela_ref[D:2 * D, :],
                      preferred_element_type=f32)
    new_relas = []
    M_objs = []
    flat_cols = []
    for b in range(PB):
        ed = edges_ref[b]
        src_col = ed[:, 0:1]
        dst_col = ed[:, 1:2]
        flat_cols.append(src_col * N + dst_col)
        M_sub = (src_col == iota_en).astype(f32)
        M_obj = (dst_col == iota_en).astype(f32)
        M_objs.append(M_obj)
        sop = jnp.dot(M_sub + M_obj, objWr1[b * N:(b + 1) * N],
                      preferred_element_type=f32)
        nr = jax.nn.relu(sop + relaWr2[b * E:(b + 1) * E]
                         + b_rela_ref[...]) + rela2[b * E:(b + 1) * E]
        new_relas.append(nr)
        rela3_ref[b] = nr

    g2 = jnp.dot(obj2, W_lin_ref[...], preferred_element_type=f32)
    ShT = (jax.lax.broadcasted_iota(jnp.int32, (D, H), 0) // HID ==
           jax.lax.broadcasted_iota(jnp.int32, (D, H), 1)).astype(f32)
    TD = (jax.lax.broadcasted_iota(jnp.int32, (D, HID), 0) % HID ==
          jax.lax.broadcasted_iota(jnp.int32, (D, HID), 1)).astype(f32)
    A1S = ShT * jnp.dot(TD, W_attn_ref[0:HID, :], preferred_element_type=f32)
    A2S = ShT * jnp.dot(TD, W_attn_ref[HID:2 * HID, :],
                        preferred_element_type=f32)
    A3S = ShT * jnp.dot(TD, W_attn_ref[2 * HID:3 * HID, :],
                        preferred_element_type=f32)
    Wl1 = jnp.dot(W_lin_ref[...], A1S, preferred_element_type=f32)
    Wl2 = jnp.dot(W_lin_ref[...], A2S, preferred_element_type=f32)
    We3 = jnp.dot(W_edge_ref[...], A3S, preferred_element_type=f32)
    s12_all = jnp.dot(obj2, jnp.concatenate([Wl1, Wl2], axis=1),
                      preferred_element_type=f32)

    Sh = (jax.lax.broadcasted_iota(jnp.int32, (H, D), 1) // HID ==
          jax.lax.broadcasted_iota(jnp.int32, (H, D), 0)).astype(f32)
    dn_rT = (((1,), (1,)), ((), ()))
    iota_ee = jax.lax.broadcasted_iota(jnp.int32, (E, E), 1)
    iota_ne = jax.lax.broadcasted_iota(jnp.int32, (N, E), 0)
    blockmask = (jax.lax.broadcasted_iota(jnp.int32, (D, D), 0) // HID ==
                 jax.lax.broadcasted_iota(jnp.int32, (D, D), 1) // HID
                 ).astype(f32)
    for b in range(PB):
        flat_col = flat_cols[b]
        flat_row = jnp.transpose(flat_col)
        g = g2[b * N:(b + 1) * N]
        s1 = s12_all[b * N:(b + 1) * N, 0:H]
        s2T = jnp.transpose(s12_all[b * N:(b + 1) * N, H:2 * H])
        s3 = jnp.dot(new_relas[b], We3, preferred_element_type=f32)

        rank = jnp.sum((flat_row < flat_col).astype(f32), axis=1,
                       keepdims=True).astype(jnp.int32)
        P = (rank == iota_ee).astype(f32)
        s3p = jnp.dot(P, s3, preferred_element_type=f32)

        R = (flat_row // N == iota_ne).astype(f32)
        C = M_objs[b]
        adj = jnp.dot(R, C, preferred_element_type=f32)

        Ct = jnp.concatenate([C] * H, axis=1)
        s3pe = jnp.dot(s3p, Sh, preferred_element_type=f32)
        s3d = jnp.dot(R, Ct * s3pe, preferred_element_type=f32)
        s1e = jnp.dot(s1, Sh, preferred_element_type=f32)
        s2cat = jnp.concatenate([s2T[h:h + 1, :] for h in range(H)],
                                axis=1)
        e_all = _leaky(s1e + s2cat + s3d)
        adjt = jnp.concatenate([adj] * H, axis=1)
        num = adjt * jnp.exp(e_all)
        den = jax.lax.dot_general(num, Sh, dn_rT,
                                  preferred_element_type=f32)
        dene = jnp.dot(den, Sh, preferred_element_type=f32)
        hasedge = jnp.sum(adj, axis=1, keepdims=True) > 0
        a_all = jnp.where(hasedge, num / dene, 1.0 / N)
        gv = jnp.concatenate([g] * H, axis=0)
        out = jnp.dot(a_all, gv * blockmask, preferred_element_type=f32)
        out_ref[b] = _leaky(out)


@jax.jit
def kernel(obj_vecs, attr_vecs, rela_vecs, edges, W_attr, b_attr,
           W_rela, b_rela, W_lin, W_edge, W_attn):
    b_attr2 = b_attr.reshape(1, D)
    b_rela2 = b_rela.reshape(1, D)

    step = lambda i: (i, 0, 0)
    const2 = lambda i: (0, 0)
    out, attr3, rela3 = pl.pallas_call(
        _fused_kernel,
        grid=(G,),
        in_specs=[
            pl.BlockSpec((PB, N, D), step),
            pl.BlockSpec((PB, N, D), step),
            pl.BlockSpec((PB, E, D), step),
            pl.BlockSpec((PB, E, 2), step),
            pl.BlockSpec((2 * D, D), const2),
            pl.BlockSpec((1, D), const2),
            pl.BlockSpec((2 * D, D), const2),
            pl.BlockSpec((1, D), const2),
            pl.BlockSpec((D, D), const2),
            pl.BlockSpec((D, D), const2),
            pl.BlockSpec((3 * HID, 1), const2),
        ],
        out_specs=[
            pl.BlockSpec((PB, N, D), step),
            pl.BlockSpec((PB, N, D), step),
            pl.BlockSpec((PB, E, D), step),
        ],
        out_shape=[
            jax.ShapeDtypeStruct((B, N, D), jnp.float32),
            jax.ShapeDtypeStruct((B, N, D), jnp.float32),
            jax.ShapeDtypeStruct((B, E, D), jnp.float32),
        ],
    )(obj_vecs, attr_vecs, rela_vecs, edges,
      W_attr, b_attr2, W_rela, b_rela2, W_lin, W_edge, W_attn)
    return (out, attr3, rela3)


# speedup vs baseline: 1.0479x; 1.0479x over previous
import jax
import jax.numpy as jnp
from jax.experimental import pallas as pl

B, N, E, D, H = 8, 64, 256, 512, 8
HID = D // H
NEG = 0.1
G = 2
PB = B // G


def _leaky(x):
    return jnp.where(x >= 0, x, NEG * x)


def _fused_kernel(obj_ref, attr_ref, rela_ref, edges_ref,
                  W_attr_ref, b_attr_ref, W_rela_ref, b_rela_ref,
                  W_lin_ref, W_edge_ref, W_attn_ref,
                  out_ref, attr3_ref, rela3_ref):
    f32 = jnp.float32
    obj2 = obj_ref[...].reshape(PB * N, D)
    attr2 = attr_ref[...].reshape(PB * N, D)
    rela2 = rela_ref[...].reshape(PB * E, D)

    new_attr = jax.nn.relu(
        jnp.dot(obj2, W_attr_ref[0:D, :], preferred_element_type=f32)
        + jnp.dot(attr2, W_attr_ref[D:2 * D, :], preferred_element_type=f32)
        + b_attr_ref[...]) + attr2
    attr3_ref[...] = new_attr.reshape(PB, N, D)

    objWr1 = jnp.dot(obj2, W_rela_ref[0:D, :],
                     preferred_element_type=f32)
    iota_en = jax.lax.broadcasted_iota(jnp.int32, (E, N), 1)
    sop_list = []
    M_objs = []
    flat_cols = []
    for b in range(PB):
        ed = edges_ref[b]
        src_col = ed[:, 0:1]
        dst_col = ed[:, 1:2]
        flat_cols.append(src_col * N + dst_col)
        M_sub = (src_col == iota_en).astype(f32)
        M_obj = (dst_col == iota_en).astype(f32)
        M_objs.append(M_obj)
        sop_list.append(jnp.dot(M_sub + M_obj, objWr1[b * N:(b + 1) * N],
                                preferred_element_type=f32))
    so_proj = jnp.concatenate(sop_list, axis=0)
    new_rela = jax.nn.relu(
        so_proj
        + jnp.dot(rela2, W_rela_ref[D:2 * D, :], preferred_element_type=f32)
        + b_rela_ref[...]) + rela2
    rela3_ref[...] = new_rela.reshape(PB, E, D)

    g2 = jnp.dot(obj2, W_lin_ref[...], preferred_element_type=f32)
    ShT = (jax.lax.broadcasted_iota(jnp.int32, (D, H), 0) // HID ==
           jax.lax.broadcasted_iota(jnp.int32, (D, H), 1)).astype(f32)
    TD = (jax.lax.broadcasted_iota(jnp.int32, (D, HID), 0) % HID ==
          jax.lax.broadcasted_iota(jnp.int32, (D, HID), 1)).astype(f32)
    A1S = ShT * jnp.dot(TD, W_attn_ref[0:HID, :], preferred_element_type=f32)
    A2S = ShT * jnp.dot(TD, W_attn_ref[HID:2 * HID, :],
                        preferred_element_type=f32)
    A3S = ShT * jnp.dot(TD, W_attn_ref[2 * HID:3 * HID, :],
                        preferred_element_type=f32)
    Wl1 = jnp.dot(W_lin_ref[...], A1S, preferred_element_type=f32)
    Wl2 = jnp.dot(W_lin_ref[...], A2S, preferred_element_type=f32)
    We3 = jnp.dot(W_edge_ref[...], A3S, preferred_element_type=f32)
    s1_all = jnp.dot(obj2, Wl1, preferred_element_type=f32)
    dn_lT = (((0,), (1,)), ((), ()))
    s2T_all = jax.lax.dot_general(Wl2, obj2, dn_lT,
                                  preferred_element_type=f32)
    s3_all = jnp.dot(new_rela, We3, preferred_element_type=f32)

    Sh = (jax.lax.broadcasted_iota(jnp.int32, (H, D), 1) // HID ==
          jax.lax.broadcasted_iota(jnp.int32, (H, D), 0)).astype(f32)
    dn_rT = (((1,), (1,)), ((), ()))
    iota_ee = jax.lax.broadcasted_iota(jnp.int32, (E, E), 1)
    iota_ne = jax.lax.broadcasted_iota(jnp.int32, (N, E), 0)
    blockmask = (jax.lax.broadcasted_iota(jnp.int32, (D, D), 0) // HID ==
                 jax.lax.broadcasted_iota(jnp.int32, (D, D), 1) // HID
                 ).astype(f32)
    for b in range(PB):
        flat_col = flat_cols[b]
        flat_row = jnp.transpose(flat_col)
        g = g2[b * N:(b + 1) * N]
        s1 = s1_all[b * N:(b + 1) * N]
        s3 = s3_all[b * E:(b + 1) * E]
        s2T = s2T_all[:, b * N:(b + 1) * N]

        rank = jnp.sum((flat_row < flat_col).astype(f32), axis=1,
                       keepdims=True).astype(jnp.int32)
        P = (rank == iota_ee).astype(f32)
        s3p = jnp.dot(P, s3, preferred_element_type=f32)

        R = (flat_row // N == iota_ne).astype(f32)
        C = M_objs[b]
        adj = jnp.dot(R, C, preferred_element_type=f32)

        Ct = jnp.concatenate([C] * H, axis=1)
        s3pe = jnp.dot(s3p, Sh, preferred_element_type=f32)
        s3d = jnp.dot(R, Ct * s3pe, preferred_element_type=f32)
        s1e = jnp.dot(s1, Sh, preferred_element_type=f32)
        s2cat = jnp.concatenate([s2T[h:h + 1, :] for h in range(H)],
                                axis=1)
        e_all = _leaky(s1e + s2cat + s3d)
        adjt = jnp.concatenate([adj] * H, axis=1)
        num = adjt * jnp.exp(e_all)
        den = jax.lax.dot_general(num, Sh, dn_rT,
                                  preferred_element_type=f32)
        dene = jnp.dot(den, Sh, preferred_element_type=f32)
        hasedge = jnp.sum(adj, axis=1, keepdims=True) > 0
        a_all = jnp.where(hasedge, num / dene, 1.0 / N)
        gv = jnp.concatenate([g] * H, axis=0)
        out = jnp.dot(a_all, gv * blockmask, preferred_element_type=f32)
        out_ref[b] = _leaky(out)


@jax.jit
def kernel(obj_vecs, attr_vecs, rela_vecs, edges, W_attr, b_attr,
           W_rela, b_rela, W_lin, W_edge, W_attn):
    b_attr2 = b_attr.reshape(1, D)
    b_rela2 = b_rela.reshape(1, D)

    step = lambda i: (i, 0, 0)
    const2 = lambda i: (0, 0)
    out, attr3, rela3 = pl.pallas_call(
        _fused_kernel,
        grid=(G,),
        in_specs=[
            pl.BlockSpec((PB, N, D), step),
            pl.BlockSpec((PB, N, D), step),
            pl.BlockSpec((PB, E, D), step),
            pl.BlockSpec((PB, E, 2), step),
            pl.BlockSpec((2 * D, D), const2),
            pl.BlockSpec((1, D), const2),
            pl.BlockSpec((2 * D, D), const2),
            pl.BlockSpec((1, D), const2),
            pl.BlockSpec((D, D), const2),
            pl.BlockSpec((D, D), const2),
            pl.BlockSpec((3 * HID, 1), const2),
        ],
        out_specs=[
            pl.BlockSpec((PB, N, D), step),
            pl.BlockSpec((PB, N, D), step),
            pl.BlockSpec((PB, E, D), step),
        ],
        out_shape=[
            jax.ShapeDtypeStruct((B, N, D), jnp.float32),
            jax.ShapeDtypeStruct((B, N, D), jnp.float32),
            jax.ShapeDtypeStruct((B, E, D), jnp.float32),
        ],
    )(obj_vecs, attr_vecs, rela_vecs, edges,
      W_attr, b_attr2, W_rela, b_rela2, W_lin, W_edge, W_attn)
    return (out, attr3, rela3)


# speedup vs baseline: 1.0660x; 1.0173x over previous
import jax
import jax.numpy as jnp
from jax.experimental import pallas as pl

B, N, E, D, H = 8, 64, 256, 512, 8
HID = D // H
NEG = 0.1
G = 1
PB = B // G


def _leaky(x):
    return jnp.where(x >= 0, x, NEG * x)


def _fused_kernel(obj_ref, attr_ref, rela_ref, edges_ref,
                  W_attr_ref, b_attr_ref, W_rela_ref, b_rela_ref,
                  W_lin_ref, W_edge_ref, W_attn_ref,
                  out_ref, attr3_ref, rela3_ref):
    f32 = jnp.float32
    obj2 = obj_ref[...].reshape(PB * N, D)
    attr2 = attr_ref[...].reshape(PB * N, D)
    rela2 = rela_ref[...].reshape(PB * E, D)

    new_attr = jax.nn.relu(
        jnp.dot(obj2, W_attr_ref[0:D, :], preferred_element_type=f32)
        + jnp.dot(attr2, W_attr_ref[D:2 * D, :], preferred_element_type=f32)
        + b_attr_ref[...]) + attr2
    attr3_ref[...] = new_attr.reshape(PB, N, D)

    objWr1 = jnp.dot(obj2, W_rela_ref[0:D, :],
                     preferred_element_type=f32)
    iota_en = jax.lax.broadcasted_iota(jnp.int32, (E, N), 1)
    sop_list = []
    M_objs = []
    flat_cols = []
    for b in range(PB):
        ed = edges_ref[b]
        src_col = ed[:, 0:1]
        dst_col = ed[:, 1:2]
        flat_cols.append(src_col * N + dst_col)
        M_sub = (src_col == iota_en).astype(f32)
        M_obj = (dst_col == iota_en).astype(f32)
        M_objs.append(M_obj)
        sop_list.append(jnp.dot(M_sub + M_obj, objWr1[b * N:(b + 1) * N],
                                preferred_element_type=f32))
    so_proj = jnp.concatenate(sop_list, axis=0)
    new_rela = jax.nn.relu(
        so_proj
        + jnp.dot(rela2, W_rela_ref[D:2 * D, :], preferred_element_type=f32)
        + b_rela_ref[...]) + rela2
    rela3_ref[...] = new_rela.reshape(PB, E, D)

    g2 = jnp.dot(obj2, W_lin_ref[...], preferred_element_type=f32)
    ShT = (jax.lax.broadcasted_iota(jnp.int32, (D, H), 0) // HID ==
           jax.lax.broadcasted_iota(jnp.int32, (D, H), 1)).astype(f32)
    TD = (jax.lax.broadcasted_iota(jnp.int32, (D, HID), 0) % HID ==
          jax.lax.broadcasted_iota(jnp.int32, (D, HID), 1)).astype(f32)
    A1S = ShT * jnp.dot(TD, W_attn_ref[0:HID, :], preferred_element_type=f32)
    A2S = ShT * jnp.dot(TD, W_attn_ref[HID:2 * HID, :],
                        preferred_element_type=f32)
    A3S = ShT * jnp.dot(TD, W_attn_ref[2 * HID:3 * HID, :],
                        preferred_element_type=f32)
    Wl1 = jnp.dot(W_lin_ref[...], A1S, preferred_element_type=f32)
    Wl2 = jnp.dot(W_lin_ref[...], A2S, preferred_element_type=f32)
    We3 = jnp.dot(W_edge_ref[...], A3S, preferred_element_type=f32)
    s1_all = jnp.dot(obj2, Wl1, preferred_element_type=f32)
    dn_lT = (((0,), (1,)), ((), ()))
    s2T_all = jax.lax.dot_general(Wl2, obj2, dn_lT,
                                  preferred_element_type=f32)
    s3_all = jnp.dot(new_rela, We3, preferred_element_type=f32)

    Sh = (jax.lax.broadcasted_iota(jnp.int32, (H, D), 1) // HID ==
          jax.lax.broadcasted_iota(jnp.int32, (H, D), 0)).astype(f32)
    dn_rT = (((1,), (1,)), ((), ()))
    iota_ee = jax.lax.broadcasted_iota(jnp.int32, (E, E), 1)
    iota_ne = jax.lax.broadcasted_iota(jnp.int32, (N, E), 0)
    blockmask = (jax.lax.broadcasted_iota(jnp.int32, (D, D), 0) // HID ==
                 jax.lax.broadcasted_iota(jnp.int32, (D, D), 1) // HID
                 ).astype(f32)
    for b in range(PB):
        flat_col = flat_cols[b]
        flat_row = jnp.transpose(flat_col)
        g = g2[b * N:(b + 1) * N]
        s1 = s1_all[b * N:(b + 1) * N]
        s3 = s3_all[b * E:(b + 1) * E]
        s2T = s2T_all[:, b * N:(b + 1) * N]

        rank = jnp.sum((flat_row < flat_col).astype(f32), axis=1,
                       keepdims=True).astype(jnp.int32)
        P = (rank == iota_ee).astype(f32)
        s3p = jnp.dot(P, s3, preferred_element_type=f32)

        R = (flat_row // N == iota_ne).astype(f32)
        C = M_objs[b]
        adj = jnp.dot(R, C, preferred_element_type=f32)

        Ct = jnp.concatenate([C] * H, axis=1)
        s3pe = jnp.dot(s3p, Sh, preferred_element_type=f32)
        s3d = jnp.dot(R, Ct * s3pe, preferred_element_type=f32)
        s1e = jnp.dot(s1, Sh, preferred_element_type=f32)
        s2cat = jnp.concatenate([s2T[h:h + 1, :] for h in range(H)],
                                axis=1)
        e_all = _leaky(s1e + s2cat + s3d)
        adjt = jnp.concatenate([adj] * H, axis=1)
        num = adjt * jnp.exp(e_all)
        den = jax.lax.dot_general(num, Sh, dn_rT,
                                  preferred_element_type=f32)
        dene = jnp.dot(den, Sh, preferred_element_type=f32)
        hasedge = jnp.sum(adj, axis=1, keepdims=True) > 0
        a_all = jnp.where(hasedge, num / dene, 1.0 / N)
        gv = jnp.concatenate([g] * H, axis=0)
        out = jnp.dot(a_all, gv * blockmask, preferred_element_type=f32)
        out_ref[b] = _leaky(out)


@jax.jit
def kernel(obj_vecs, attr_vecs, rela_vecs, edges, W_attr, b_attr,
           W_rela, b_rela, W_lin, W_edge, W_attn):
    b_attr2 = b_attr.reshape(1, D)
    b_rela2 = b_rela.reshape(1, D)

    step = lambda i: (i, 0, 0)
    const2 = lambda i: (0, 0)
    out, attr3, rela3 = pl.pallas_call(
        _fused_kernel,
        grid=(G,),
        in_specs=[
            pl.BlockSpec((PB, N, D), step),
            pl.BlockSpec((PB, N, D), step),
            pl.BlockSpec((PB, E, D), step),
            pl.BlockSpec((PB, E, 2), step),
            pl.BlockSpec((2 * D, D), const2),
            pl.BlockSpec((1, D), const2),
            pl.BlockSpec((2 * D, D), const2),
            pl.BlockSpec((1, D), const2),
            pl.BlockSpec((D, D), const2),
            pl.BlockSpec((D, D), const2),
            pl.BlockSpec((3 * HID, 1), const2),
        ],
        out_specs=[
            pl.BlockSpec((PB, N, D), step),
            pl.BlockSpec((PB, N, D), step),
            pl.BlockSpec((PB, E, D), step),
        ],
        out_shape=[
            jax.ShapeDtypeStruct((B, N, D), jnp.float32),
            jax.ShapeDtypeStruct((B, N, D), jnp.float32),
            jax.ShapeDtypeStruct((B, E, D), jnp.float32),
        ],
    )(obj_vecs, attr_vecs, rela_vecs, edges,
      W_attr, b_attr2, W_rela, b_rela2, W_lin, W_edge, W_attn)
    return (out, attr3, rela3)


# speedup vs baseline: 1.0729x; 1.0065x over previous
import jax
import jax.numpy as jnp
from jax.experimental import pallas as pl

B, N, E, D, H = 8, 64, 256, 512, 8
HID = D // H
NEG = 0.1
G = 1
PB = B // G


def _leaky(x):
    return jnp.where(x >= 0, x, NEG * x)


def _fused_kernel(obj_ref, attr_ref, rela_ref, edges_ref,
                  W_attr_ref, b_attr_ref, W_rela_ref, b_rela_ref,
                  W_lin_ref, W_edge_ref, W_attn_ref,
                  out_ref, attr3_ref, rela3_ref):
    f32 = jnp.float32
    obj2 = obj_ref[...].reshape(PB * N, D)
    attr2 = attr_ref[...].reshape(PB * N, D)
    rela2 = rela_ref[...].reshape(PB * E, D)

    new_attr = jax.nn.relu(
        jnp.dot(obj2, W_attr_ref[0:D, :], preferred_element_type=f32)
        + jnp.dot(attr2, W_attr_ref[D:2 * D, :], preferred_element_type=f32)
        + b_attr_ref[...]) + attr2
    attr3_ref[...] = new_attr.reshape(PB, N, D)

    objWr1 = jnp.dot(obj2, W_rela_ref[0:D, :],
                     preferred_element_type=f32)
    iota_en = jax.lax.broadcasted_iota(jnp.int32, (E, N), 1)
    sop_list = []
    M_objs = []
    flat_cols = []
    for b in range(PB):
        ed = edges_ref[b]
        src_col = ed[:, 0:1]
        dst_col = ed[:, 1:2]
        flat_cols.append(src_col * N + dst_col)
        M_sub = (src_col == iota_en).astype(f32)
        M_obj = (dst_col == iota_en).astype(f32)
        M_objs.append(M_obj)
        sop_list.append(jnp.dot(M_sub + M_obj, objWr1[b * N:(b + 1) * N],
                                preferred_element_type=f32))
    so_proj = jnp.concatenate(sop_list, axis=0)
    new_rela = jax.nn.relu(
        so_proj
        + jnp.dot(rela2, W_rela_ref[D:2 * D, :], preferred_element_type=f32)
        + b_rela_ref[...]) + rela2
    rela3_ref[...] = new_rela.reshape(PB, E, D)

    g2 = jnp.dot(obj2, W_lin_ref[...], preferred_element_type=f32)
    ShT = (jax.lax.broadcasted_iota(jnp.int32, (D, H), 0) // HID ==
           jax.lax.broadcasted_iota(jnp.int32, (D, H), 1)).astype(f32)
    TD = (jax.lax.broadcasted_iota(jnp.int32, (D, HID), 0) % HID ==
          jax.lax.broadcasted_iota(jnp.int32, (D, HID), 1)).astype(f32)
    A1S = ShT * jnp.dot(TD, W_attn_ref[0:HID, :], preferred_element_type=f32)
    A2S = ShT * jnp.dot(TD, W_attn_ref[HID:2 * HID, :],
                        preferred_element_type=f32)
    A3S = ShT * jnp.dot(TD, W_attn_ref[2 * HID:3 * HID, :],
                        preferred_element_type=f32)
    Wl12 = jnp.dot(W_lin_ref[...], jnp.concatenate([A1S, A2S], axis=1),
                   preferred_element_type=f32)
    We3 = jnp.dot(W_edge_ref[...], A3S, preferred_element_type=f32)
    s12_all = jnp.dot(obj2, Wl12, preferred_element_type=f32)
    s3_all = jnp.dot(new_rela, We3, preferred_element_type=f32)

    Sh = (jax.lax.broadcasted_iota(jnp.int32, (H, D), 1) // HID ==
          jax.lax.broadcasted_iota(jnp.int32, (H, D), 0)).astype(f32)
    dn_rT = (((1,), (1,)), ((), ()))
    iota_ee = jax.lax.broadcasted_iota(jnp.int32, (E, E), 1)
    iota_ne = jax.lax.broadcasted_iota(jnp.int32, (N, E), 0)
    blockmask = (jax.lax.broadcasted_iota(jnp.int32, (D, D), 0) // HID ==
                 jax.lax.broadcasted_iota(jnp.int32, (D, D), 1) // HID
                 ).astype(f32)
    for b in range(PB):
        flat_col = flat_cols[b]
        flat_row = jnp.transpose(flat_col)
        g = g2[b * N:(b + 1) * N]
        s1 = s12_all[b * N:(b + 1) * N, 0:H]
        s3 = s3_all[b * E:(b + 1) * E]
        s2T = jnp.transpose(s12_all[b * N:(b + 1) * N, H:2 * H])

        rank = jnp.sum((flat_row < flat_col).astype(f32), axis=1,
                       keepdims=True).astype(jnp.int32)
        P = (rank == iota_ee).astype(f32)
        s3p = jnp.dot(P, s3, preferred_element_type=f32)

        R = (flat_row // N == iota_ne).astype(f32)
        C = M_objs[b]
        adj = jnp.dot(R, C, preferred_element_type=f32)

        Ct = jnp.concatenate([C] * H, axis=1)
        s3pe = jnp.dot(s3p, Sh, preferred_element_type=f32)
        s3d = jnp.dot(R, Ct * s3pe, preferred_element_type=f32)
        s1e = jnp.dot(s1, Sh, preferred_element_type=f32)
        s2cat = jnp.concatenate([s2T[h:h + 1, :] for h in range(H)],
                                axis=1)
        e_all = _leaky(s1e + s2cat + s3d)
        adjt = jnp.concatenate([adj] * H, axis=1)
        num = adjt * jnp.exp(e_all)
        den = jax.lax.dot_general(num, Sh, dn_rT,
                                  preferred_element_type=f32)
        dene = jnp.dot(den, Sh, preferred_element_type=f32)
        hasedge = jnp.sum(adj, axis=1, keepdims=True) > 0
        a_all = jnp.where(hasedge, num / dene, 1.0 / N)
        gv = jnp.concatenate([g] * H, axis=0)
        out = jnp.dot(a_all, gv * blockmask, preferred_element_type=f32)
        out_ref[b] = _leaky(out)


@jax.jit
def kernel(obj_vecs, attr_vecs, rela_vecs, edges, W_attr, b_attr,
           W_rela, b_rela, W_lin, W_edge, W_attn):
    b_attr2 = b_attr.reshape(1, D)
    b_rela2 = b_rela.reshape(1, D)

    step = lambda i: (i, 0, 0)
    const2 = lambda i: (0, 0)
    out, attr3, rela3 = pl.pallas_call(
        _fused_kernel,
        grid=(G,),
        in_specs=[
            pl.BlockSpec((PB, N, D), step),
            pl.BlockSpec((PB, N, D), step),
            pl.BlockSpec((PB, E, D), step),
            pl.BlockSpec((PB, E, 2), step),
            pl.BlockSpec((2 * D, D), const2),
            pl.BlockSpec((1, D), const2),
            pl.BlockSpec((2 * D, D), const2),
            pl.BlockSpec((1, D), const2),
            pl.BlockSpec((D, D), const2),
            pl.BlockSpec((D, D), const2),
            pl.BlockSpec((3 * HID, 1), const2),
        ],
        out_specs=[
            pl.BlockSpec((PB, N, D), step),
            pl.BlockSpec((PB, N, D), step),
            pl.BlockSpec((PB, E, D), step),
        ],
        out_shape=[
            jax.ShapeDtypeStruct((B, N, D), jnp.float32),
            jax.ShapeDtypeStruct((B, N, D), jnp.float32),
            jax.ShapeDtypeStruct((B, E, D), jnp.float32),
        ],
    )(obj_vecs, attr_vecs, rela_vecs, edges,
      W_attr, b_attr2, W_rela, b_rela2, W_lin, W_edge, W_attn)
    return (out, attr3, rela3)
